# baseline (speedup 1.0000x reference)
import functools

import jax
import jax.numpy as jnp
from jax import lax
from jax.experimental import pallas as pl
from jax.experimental.pallas import tpu as pltpu
from jax.experimental.pallas import tpu_sc as plsc

DIM_IN = 1024
DIM_OUT = 1024
NUM_GROUPS = 8
TOKENS = 8192
BLK = 128
NB = TOKENS // BLK + NUM_GROUPS
P = NB * BLK
OUT_PAD = P


@functools.cache
def _make_sc_scatter(n_src, n_dst, chunk, nbuf=3):
    info = plsc.get_sparse_core_info()
    nc, ns = info.num_cores, info.num_subcores
    nw = nc * ns
    rows_pw = n_src // nw
    n_chunks = rows_pw // chunk
    assert rows_pw % chunk == 0 and chunk % 8 == 0 and chunk <= 128

    def body(src_hbm, idx_hbm, out_hbm, idx_v, *rest):
        bufs = rest[:nbuf]
        rsems = rest[nbuf:2 * nbuf]
        wsems = rest[2 * nbuf:3 * nbuf]
        wid = lax.axis_index("s") * nc + lax.axis_index("c")
        base = wid * rows_pw
        pltpu.sync_copy(idx_hbm.at[wid], idx_v)
        rdh = [None] * nbuf
        wrh = [None] * nbuf
        for c in range(n_chunks):
            i = c % nbuf
            if c >= nbuf:
                wrh[i].wait()
            rdh[i] = pltpu.async_copy(
                src_hbm.at[pl.ds(base + c * chunk, chunk)], bufs[i], rsems[i])
            if c >= nbuf - 1:
                d = c - (nbuf - 1)
                j = d % nbuf
                rdh[j].wait()
                wrh[j] = pltpu.async_copy(
                    bufs[j], out_hbm.at[idx_v.at[d]], wsems[j])
        for d in range(max(n_chunks - nbuf + 1, 0), n_chunks):
            j = d % nbuf
            rdh[j].wait()
            wrh[j] = pltpu.async_copy(
                bufs[j], out_hbm.at[idx_v.at[d]], wsems[j])
        for d in range(max(n_chunks - nbuf, 0), n_chunks):
            wrh[d % nbuf].wait()

    return pl.kernel(
        body,
        out_type=jax.ShapeDtypeStruct((n_dst, DIM_IN), jnp.float32),
        mesh=plsc.VectorSubcoreMesh(core_axis_name="c", subcore_axis_name="s"),
        scratch_types=[pltpu.VMEM((n_chunks, chunk), jnp.int32)]
        + [pltpu.VMEM((chunk, DIM_IN), jnp.float32) for _ in range(nbuf)]
        + [pltpu.SemaphoreType.DMA for _ in range(2 * nbuf)],
    )


def _mm_body(widx_ref, x_ref, w_ref, b_ref, o_ref):
    o_ref[...] = jax.lax.dot_general(
        x_ref[...], w_ref[0],
        (((1,), (1,)), ((), ())),
        preferred_element_type=jnp.float32,
    ) + b_ref[0, 0][None, :]


def _tc_group_matmul(w_idx, xs, W3, b3):
    grid_spec = pltpu.PrefetchScalarGridSpec(
        num_scalar_prefetch=1,
        grid=(NB,),
        in_specs=[
            pl.BlockSpec((BLK, DIM_IN), lambda i, r: (i, 0)),
            pl.BlockSpec((1, DIM_OUT, DIM_IN), lambda i, r: (r[i], 0, 0)),
            pl.BlockSpec((1, 1, DIM_OUT), lambda i, r: (r[i], 0, 0)),
        ],
        out_specs=pl.BlockSpec((BLK, DIM_OUT), lambda i, r: (i, 0)),
    )
    return pl.pallas_call(
        _mm_body,
        grid_spec=grid_spec,
        out_shape=jax.ShapeDtypeStruct((P, DIM_OUT), jnp.float32),
        compiler_params=pltpu.CompilerParams(
            dimension_semantics=("parallel",),
        ),
    )(w_idx, xs, W3, b3)


def _route(g):
    onehot = (g[:, None] == jnp.arange(NUM_GROUPS, dtype=jnp.int32)[None, :])
    cum = jnp.cumsum(onehot.astype(jnp.int32), axis=0)
    counts = cum[-1]
    r_t = jnp.take_along_axis(cum, g[:, None], axis=1)[:, 0] - 1
    present = counts > 0
    rank = jnp.cumsum(present.astype(jnp.int32)) - 1
    padded = ((counts + BLK - 1) // BLK) * BLK
    pad_end = jnp.cumsum(padded)
    pad_start = pad_end - padded
    pos = pad_start[g] + r_t
    gather_idx = (jnp.arange(P, dtype=jnp.int32) + TOKENS).at[pos].set(
        jnp.arange(TOKENS, dtype=jnp.int32), unique_indices=True)
    blk_lo = jnp.arange(NB, dtype=jnp.int32) * BLK
    blk_gid = jnp.minimum(
        jnp.sum((blk_lo[:, None] >= pad_end[None, :]).astype(jnp.int32), axis=1),
        NUM_GROUPS - 1)
    w_idx = jnp.maximum(rank[blk_gid], 0)
    return gather_idx, w_idx, pos


@jax.jit
def kernel(x, group_by, W, b):
    g = group_by.astype(jnp.int32)
    gather_idx, w_idx, pos = _route(g)

    W3 = W.reshape(NUM_GROUPS, DIM_OUT, DIM_IN)
    b3 = b.reshape(NUM_GROUPS, 1, DIM_OUT)

    nw = 32
    pos_t = pos.reshape(nw, -1, 32)
    inv_t = gather_idx.reshape(nw, -1, 32)

    xs = _make_sc_scatter(TOKENS, P, 32)(x, pos_t)
    ys = _tc_group_matmul(w_idx, xs, W3, b3)
    full = _make_sc_scatter(P, TOKENS + OUT_PAD, 32)(ys, inv_t)
    return full[:TOKENS]


# speedup vs baseline: 1.1294x; 1.1294x over previous
import functools

import jax
import jax.numpy as jnp
from jax import lax
from jax.experimental import pallas as pl
from jax.experimental.pallas import tpu as pltpu
from jax.experimental.pallas import tpu_sc as plsc

DIM_IN = 1024
DIM_OUT = 1024
NUM_GROUPS = 8
TOKENS = 8192
BLK = 512
NB = TOKENS // BLK + NUM_GROUPS
P = NB * BLK
OUT_PAD = P


@functools.cache
def _make_sc_scatter(n_src, n_dst, chunk, nbuf=3):
    info = plsc.get_sparse_core_info()
    nc, ns = info.num_cores, info.num_subcores
    nw = nc * ns
    rows_pw = n_src // nw
    n_chunks = rows_pw // chunk
    assert rows_pw % chunk == 0 and chunk % 8 == 0 and chunk <= 128

    def body(src_hbm, idx_hbm, out_hbm, idx_v, *rest):
        bufs = rest[:nbuf]
        rsems = rest[nbuf:2 * nbuf]
        wsems = rest[2 * nbuf:3 * nbuf]
        wid = lax.axis_index("s") * nc + lax.axis_index("c")
        base = wid * rows_pw
        pltpu.sync_copy(idx_hbm.at[wid], idx_v)
        rdh = [None] * nbuf
        wrh = [None] * nbuf
        for c in range(n_chunks):
            i = c % nbuf
            if c >= nbuf:
                wrh[i].wait()
            rdh[i] = pltpu.async_copy(
                src_hbm.at[pl.ds(base + c * chunk, chunk)], bufs[i], rsems[i])
            if c >= nbuf - 1:
                d = c - (nbuf - 1)
                j = d % nbuf
                rdh[j].wait()
                wrh[j] = pltpu.async_copy(
                    bufs[j], out_hbm.at[idx_v.at[d]], wsems[j])
        for d in range(max(n_chunks - nbuf + 1, 0), n_chunks):
            j = d % nbuf
            rdh[j].wait()
            wrh[j] = pltpu.async_copy(
                bufs[j], out_hbm.at[idx_v.at[d]], wsems[j])
        for d in range(max(n_chunks - nbuf, 0), n_chunks):
            wrh[d % nbuf].wait()

    return pl.kernel(
        body,
        out_type=jax.ShapeDtypeStruct((n_dst, DIM_IN), jnp.float32),
        mesh=plsc.VectorSubcoreMesh(core_axis_name="c", subcore_axis_name="s"),
        scratch_types=[pltpu.VMEM((n_chunks, chunk), jnp.int32)]
        + [pltpu.VMEM((chunk, DIM_IN), jnp.float32) for _ in range(nbuf)]
        + [pltpu.SemaphoreType.DMA for _ in range(2 * nbuf)],
    )


def _mm_body(widx_ref, x_ref, w_ref, b_ref, o_ref):
    o_ref[...] = jax.lax.dot_general(
        x_ref[...], w_ref[0],
        (((1,), (1,)), ((), ())),
        preferred_element_type=jnp.float32,
    ) + b_ref[0, 0][None, :]


def _tc_group_matmul(w_idx, xs, W3, b3):
    grid_spec = pltpu.PrefetchScalarGridSpec(
        num_scalar_prefetch=1,
        grid=(NB,),
        in_specs=[
            pl.BlockSpec((BLK, DIM_IN), lambda i, r: (i, 0)),
            pl.BlockSpec((1, DIM_OUT, DIM_IN), lambda i, r: (r[i], 0, 0)),
            pl.BlockSpec((1, 1, DIM_OUT), lambda i, r: (r[i], 0, 0)),
        ],
        out_specs=pl.BlockSpec((BLK, DIM_OUT), lambda i, r: (i, 0)),
    )
    return pl.pallas_call(
        _mm_body,
        grid_spec=grid_spec,
        out_shape=jax.ShapeDtypeStruct((P, DIM_OUT), jnp.float32),
        compiler_params=pltpu.CompilerParams(
            dimension_semantics=("parallel",),
        ),
    )(w_idx, xs, W3, b3)


def _route(g):
    onehot = (g[:, None] == jnp.arange(NUM_GROUPS, dtype=jnp.int32)[None, :])
    cum = jnp.cumsum(onehot.astype(jnp.int32), axis=0)
    counts = cum[-1]
    r_t = jnp.take_along_axis(cum, g[:, None], axis=1)[:, 0] - 1
    present = counts > 0
    rank = jnp.cumsum(present.astype(jnp.int32)) - 1
    padded = ((counts + BLK - 1) // BLK) * BLK
    pad_end = jnp.cumsum(padded)
    pad_start = pad_end - padded
    pos = pad_start[g] + r_t
    gather_idx = (jnp.arange(P, dtype=jnp.int32) + TOKENS).at[pos].set(
        jnp.arange(TOKENS, dtype=jnp.int32), unique_indices=True)
    blk_lo = jnp.arange(NB, dtype=jnp.int32) * BLK
    blk_gid = jnp.minimum(
        jnp.sum((blk_lo[:, None] >= pad_end[None, :]).astype(jnp.int32), axis=1),
        NUM_GROUPS - 1)
    w_idx = jnp.maximum(rank[blk_gid], 0)
    return gather_idx, w_idx, pos


@jax.jit
def kernel(x, group_by, W, b):
    g = group_by.astype(jnp.int32)
    gather_idx, w_idx, pos = _route(g)

    W3 = W.reshape(NUM_GROUPS, DIM_OUT, DIM_IN)
    b3 = b.reshape(NUM_GROUPS, 1, DIM_OUT)

    nw = 32
    pos_t = pos.reshape(nw, -1, 32)
    inv_t = gather_idx.reshape(nw, -1, 32)

    xs = _make_sc_scatter(TOKENS, P, 32)(x, pos_t)
    ys = _tc_group_matmul(w_idx, xs, W3, b3)
    full = _make_sc_scatter(P, TOKENS + OUT_PAD, 32)(ys, inv_t)
    return full[:TOKENS]
